# baseline (speedup 1.0000x reference)
import functools

import jax
import jax.numpy as jnp
from jax import lax
from jax.experimental import pallas as pl
from jax.experimental.pallas import tpu as pltpu
from jax.experimental.pallas import tpu_sc as plsc

_VOCAB = 100000
_DIM = 300
_DP = 384
_B = 1024
_L = 20
_MAX_NORM = 1.0

_NC = 2
_NS = 16
_NW = _NC * _NS
_R = _B * _L
_R_PER_W = _R // _NW
_GC = 128
_CPW = _R_PER_W // _GC

_TB = 2048
_TGRID = (_VOCAB + _TB - 1) // _TB


def _tc_transpose(wt):
    def body(w_ref, out_ref):
        eye = (
            lax.broadcasted_iota(jnp.int32, (_DIM, _DP), 0)
            == lax.broadcasted_iota(jnp.int32, (_DIM, _DP), 1)
        ).astype(jnp.bfloat16)
        wb = w_ref[...].astype(jnp.bfloat16)
        out_ref[...] = lax.dot_general(
            wb, eye, (((0,), (0,)), ((), ())),
            preferred_element_type=jnp.float32,
        )

    return pl.pallas_call(
        body,
        grid=(_TGRID,),
        in_specs=[pl.BlockSpec((_DIM, _TB), lambda j: (0, j))],
        out_specs=pl.BlockSpec((_TB, _DP), lambda j: (j, 0)),
        out_shape=jax.ShapeDtypeStruct((_VOCAB, _DP), jnp.float32),
    )(wt)


def _sc_gather(idx3d, table384):
    mesh = plsc.VectorSubcoreMesh(core_axis_name="c", subcore_axis_name="s")

    @functools.partial(
        pl.kernel,
        out_type=jax.ShapeDtypeStruct((_R, _DP), jnp.float32),
        mesh=mesh,
        compiler_params=pltpu.CompilerParams(
            use_tc_tiling_on_sc=True, needs_layout_passes=False
        ),
        scratch_types=[
            pltpu.VMEM((8, _GC), jnp.int32),
            pltpu.VMEM((_GC, _DP), jnp.float32),
            pltpu.SemaphoreType.DMA,
        ],
    )
    def gather_kernel(idx_hbm, t_hbm, out_hbm, idx_v, buf, sem):
        wid = lax.axis_index("s") * _NC + lax.axis_index("c")
        base = wid * _R_PER_W
        pltpu.sync_copy(idx_hbm.at[pl.ds(wid * 8, 8)], idx_v)
        for c in range(_CPW):
            pltpu.async_copy(t_hbm.at[idx_v.at[c]], buf, sem).wait()
            pltpu.sync_copy(buf, out_hbm.at[pl.ds(base + c * _GC, _GC)])

    return gather_kernel(idx3d, table384)


_VB = 1024
_NBLK = (_VOCAB + _VB - 1) // _VB
_GRID = _L + _NBLK


def _tc_pool_matmul(rows, wt, bias2):
    def body(rows_ref, w_ref, b_ref, out_ref, xbf_ref, xf_ref):
        j = pl.program_id(0)

        @pl.when(j < _L)
        def _():
            r = rows_ref[...]
            ss = jnp.sum(r * r, axis=1, keepdims=True)
            scale = jnp.where(
                ss > _MAX_NORM * _MAX_NORM,
                _MAX_NORM / (jnp.sqrt(ss) + 1e-7),
                1.0,
            )
            contrib = r * scale

            @pl.when(j == 0)
            def _():
                xf_ref[...] = contrib

            @pl.when(j > 0)
            def _():
                xf_ref[...] += contrib

            @pl.when(j == _L - 1)
            def _():
                xbf_ref[...] = (
                    (xf_ref[...] * (1.0 / _L))[:, :_DIM].astype(jnp.bfloat16)
                )

        @pl.when(j >= _L)
        def _():
            wb = w_ref[...].astype(jnp.bfloat16)
            outt = lax.dot_general(
                wb, xbf_ref[...], (((0,), (1,)), ((), ())),
                preferred_element_type=jnp.float32,
            )
            out_ref[...] = outt + jnp.transpose(b_ref[...], (1, 0))

    return pl.pallas_call(
        body,
        grid=(_GRID,),
        in_specs=[
            pl.BlockSpec((_B, _DP), lambda j: (jnp.minimum(j, _L - 1), 0)),
            pl.BlockSpec((_DIM, _VB), lambda j: (0, jnp.maximum(j - _L, 0))),
            pl.BlockSpec((1, _VB), lambda j: (0, jnp.maximum(j - _L, 0))),
        ],
        out_specs=pl.BlockSpec((_VB, _B), lambda j: (jnp.maximum(j - _L, 0), 0)),
        out_shape=jax.ShapeDtypeStruct((_VOCAB, _B), jnp.float32),
        scratch_shapes=[
            pltpu.VMEM((_B, _DIM), jnp.bfloat16),
            pltpu.VMEM((_B, _DP), jnp.float32),
        ],
    )(rows, wt, bias2)


def kernel(inputs_, embedding_weight, linear_weight, linear_bias):
    idx = jnp.transpose(inputs_).reshape(_NW, _CPW, _GC).astype(jnp.int32)
    idx3d = jnp.pad(idx, ((0, 0), (0, 8 - _CPW), (0, 0))).reshape(_NW * 8, _GC)
    table384 = _tc_transpose(embedding_weight.T)
    rows = _sc_gather(idx3d, table384)
    bias2 = linear_bias.reshape(1, _VOCAB)
    outt = _tc_pool_matmul(rows, linear_weight.T, bias2)
    return outt.T


# speedup vs baseline: 1.0143x; 1.0143x over previous
import functools

import jax
import jax.numpy as jnp
from jax import lax
from jax.experimental import pallas as pl
from jax.experimental.pallas import tpu as pltpu
from jax.experimental.pallas import tpu_sc as plsc

_VOCAB = 100000
_DIM = 300
_DP = 384
_B = 1024
_L = 20
_MAX_NORM = 1.0

_NC = 2
_NS = 16
_NW = _NC * _NS
_R = _B * _L
_R_PER_W = _R // _NW
_GC = 128
_CPW = _R_PER_W // _GC

_TB = 2048
_TGRID = (_VOCAB + _TB - 1) // _TB


def _tc_transpose(wt):
    def body(w_ref, out_ref):
        out_ref[:, :_DIM] = jnp.transpose(w_ref[...], (1, 0))
        out_ref[:, _DIM:] = jnp.zeros((_TB, _DP - _DIM), jnp.float32)

    return pl.pallas_call(
        body,
        grid=(_TGRID,),
        in_specs=[pl.BlockSpec((_DIM, _TB), lambda j: (0, j))],
        out_specs=pl.BlockSpec((_TB, _DP), lambda j: (j, 0)),
        out_shape=jax.ShapeDtypeStruct((_VOCAB, _DP), jnp.float32),
    )(wt)


def _sc_gather(idx3d, table384):
    mesh = plsc.VectorSubcoreMesh(core_axis_name="c", subcore_axis_name="s")

    @functools.partial(
        pl.kernel,
        out_type=jax.ShapeDtypeStruct((_R, _DP), jnp.float32),
        mesh=mesh,
        compiler_params=pltpu.CompilerParams(
            use_tc_tiling_on_sc=True, needs_layout_passes=False
        ),
        scratch_types=[
            pltpu.VMEM((8, _GC), jnp.int32),
            pltpu.VMEM((_GC, _DP), jnp.float32),
            pltpu.SemaphoreType.DMA,
        ],
    )
    def gather_kernel(idx_hbm, t_hbm, out_hbm, idx_v, buf, sem):
        wid = lax.axis_index("s") * _NC + lax.axis_index("c")
        base = wid * _R_PER_W
        pltpu.sync_copy(idx_hbm.at[pl.ds(wid * 8, 8)], idx_v)
        for c in range(_CPW):
            pltpu.async_copy(t_hbm.at[idx_v.at[c]], buf, sem).wait()
            pltpu.sync_copy(buf, out_hbm.at[pl.ds(base + c * _GC, _GC)])

    return gather_kernel(idx3d, table384)


_VB = 1024
_KA = 304
_NBLK = (_VOCAB + _VB - 1) // _VB
_GRID = _L + _NBLK


def _tc_pool_matmul(rows, wt, bias2):
    def body(rows_ref, w_ref, b_ref, out_ref, xbf_ref, xf_ref, wa_ref):
        j = pl.program_id(0)

        @pl.when(j < _L)
        def _():
            r = rows_ref[...]
            ss = jnp.sum(r * r, axis=1, keepdims=True)
            scale = jnp.where(
                ss > _MAX_NORM * _MAX_NORM,
                _MAX_NORM / (jnp.sqrt(ss) + 1e-7),
                1.0,
            )
            contrib = r * scale

            @pl.when(j == 0)
            def _():
                xf_ref[...] = contrib

            @pl.when(j > 0)
            def _():
                xf_ref[...] += contrib

            @pl.when(j == _L - 1)
            def _():
                xbf_ref[:, :_DIM] = (
                    (xf_ref[...] * (1.0 / _L))[:, :_DIM].astype(jnp.bfloat16)
                )
                xbf_ref[:, _DIM:] = jnp.concatenate(
                    [
                        jnp.ones((_B, 1), jnp.bfloat16),
                        jnp.zeros((_B, _KA - _DIM - 1), jnp.bfloat16),
                    ],
                    axis=1,
                )

        @pl.when(j >= _L)
        def _():
            wa_ref[:_DIM, :] = w_ref[...].astype(jnp.bfloat16)
            wa_ref[_DIM:_DIM + 1, :] = b_ref[...].astype(jnp.bfloat16)

            @pl.when(j == _L)
            def _():
                wa_ref[_DIM + 1:, :] = jnp.zeros((_KA - _DIM - 1, _VB), jnp.bfloat16)

            out_ref[...] = lax.dot_general(
                wa_ref[...], xbf_ref[...], (((0,), (1,)), ((), ())),
                preferred_element_type=jnp.float32,
            )

    return pl.pallas_call(
        body,
        grid=(_GRID,),
        in_specs=[
            pl.BlockSpec((_B, _DP), lambda j: (jnp.minimum(j, _L - 1), 0)),
            pl.BlockSpec((_DIM, _VB), lambda j: (0, jnp.maximum(j - _L, 0))),
            pl.BlockSpec((1, _VB), lambda j: (0, jnp.maximum(j - _L, 0))),
        ],
        out_specs=pl.BlockSpec((_VB, _B), lambda j: (jnp.maximum(j - _L, 0), 0)),
        out_shape=jax.ShapeDtypeStruct((_VOCAB, _B), jnp.float32),
        scratch_shapes=[
            pltpu.VMEM((_B, _KA), jnp.bfloat16),
            pltpu.VMEM((_B, _DP), jnp.float32),
            pltpu.VMEM((_KA, _VB), jnp.bfloat16),
        ],
    )(rows, wt, bias2)


def kernel(inputs_, embedding_weight, linear_weight, linear_bias):
    idx = jnp.transpose(inputs_).reshape(_NW, _CPW, _GC).astype(jnp.int32)
    idx3d = jnp.pad(idx, ((0, 0), (0, 8 - _CPW), (0, 0))).reshape(_NW * 8, _GC)
    table384 = _tc_transpose(embedding_weight.T)
    rows = _sc_gather(idx3d, table384)
    bias2 = linear_bias.reshape(1, _VOCAB)
    outt = _tc_pool_matmul(rows, linear_weight.T, bias2)
    return outt.T
